# baseline (speedup 1.0000x reference)
import functools

import jax
import jax.numpy as jnp
from jax import lax
from jax.experimental import pallas as pl
from jax.experimental.pallas import tpu as pltpu
from jax.experimental.pallas import tpu_sc as plsc

DIM = 128
CHUNK = 128
NC, NS = 2, 16
NW = NC * NS


@functools.lru_cache(maxsize=None)
def _gather_fn(n_per_w: int):
    mesh = plsc.VectorSubcoreMesh(core_axis_name="c", subcore_axis_name="s")

    NBUF = 4

    @functools.partial(
        pl.kernel,
        mesh=mesh,
        out_type=jax.ShapeDtypeStruct((NW * n_per_w * CHUNK, DIM), jnp.float32),
        scratch_types=[pltpu.VMEM((n_per_w, CHUNK), jnp.int32)]
        + [pltpu.VMEM((CHUNK, DIM), jnp.float32)] * NBUF
        + [pltpu.SemaphoreType.DMA] * (2 * NBUF),
    )
    def k(idx_hbm, table_hbm, out_hbm, idx_v, *bufs):
        rows = bufs[:NBUF]
        gs = bufs[NBUF:2 * NBUF]
        ws = bufs[2 * NBUF:]
        wid = lax.axis_index("s") * NC + lax.axis_index("c")
        cbase = wid * n_per_w
        pltpu.sync_copy(idx_hbm.at[pl.ds(cbase, n_per_w)], idx_v)

        def out_slice(j):
            return out_hbm.at[pl.ds((cbase + j) * CHUNK, CHUNK)]

        for b in range(NBUF - 1):
            pltpu.async_copy(table_hbm.at[idx_v.at[b]], rows[b], gs[b])

        ngrp = n_per_w // NBUF

        def body(g, carry):
            for b in range(NBUF):
                j = g * NBUF + b
                bp = (b + NBUF - 1) % NBUF
                pltpu.make_async_copy(
                    table_hbm.at[idx_v.at[j]], rows[b], gs[b]).wait()
                pltpu.async_copy(rows[b], out_slice(j), ws[b])

                @pl.when(j + NBUF - 1 < n_per_w)
                def _():
                    @pl.when(j >= 1)
                    def _w():
                        pltpu.make_async_copy(
                            rows[bp], out_slice(j - 1), ws[bp]).wait()

                    pltpu.async_copy(
                        table_hbm.at[idx_v.at[j + NBUF - 1]], rows[bp], gs[bp])

            return carry

        lax.fori_loop(0, ngrp, body, 0)

        for b in range(NBUF):
            j = n_per_w - NBUF + b
            pltpu.make_async_copy(rows[b], out_slice(j), ws[b]).wait()

    return k


def kernel(input, table):
    flat = input.reshape(-1).astype(jnp.int32)
    n_chunks = flat.shape[0] // CHUNK
    idx2d = flat.reshape(n_chunks, CHUNK)
    out = _gather_fn(n_chunks // NW)(idx2d, table)
    return out.reshape(input.shape + (DIM,))


# speedup vs baseline: 1.0013x; 1.0013x over previous
import functools

import jax
import jax.numpy as jnp
from jax import lax
from jax.experimental import pallas as pl
from jax.experimental.pallas import tpu as pltpu
from jax.experimental.pallas import tpu_sc as plsc

DIM = 128
CHUNK = 128
NC, NS = 2, 16
NW = NC * NS


@functools.lru_cache(maxsize=None)
def _gather_fn(n_per_w: int):
    mesh = plsc.VectorSubcoreMesh(core_axis_name="c", subcore_axis_name="s")

    NBUF = 4

    @functools.partial(
        pl.kernel,
        mesh=mesh,
        out_type=jax.ShapeDtypeStruct((NW * n_per_w * CHUNK, DIM), jnp.float32),
        scratch_types=[pltpu.VMEM((n_per_w * CHUNK,), jnp.int32)]
        + [pltpu.VMEM((CHUNK, DIM), jnp.float32)] * NBUF
        + [pltpu.SemaphoreType.DMA] * (2 * NBUF),
    )
    def k(idx_hbm, table_hbm, out_hbm, idx_v, *bufs):
        rows = bufs[:NBUF]
        gs = bufs[NBUF:2 * NBUF]
        ws = bufs[2 * NBUF:]
        wid = lax.axis_index("s") * NC + lax.axis_index("c")
        cbase = wid * n_per_w
        pltpu.sync_copy(idx_hbm.at[pl.ds(cbase * CHUNK, n_per_w * CHUNK)], idx_v)

        def out_slice(j):
            return out_hbm.at[pl.ds((cbase + j) * CHUNK, CHUNK)]

        for b in range(NBUF - 1):
            pltpu.async_copy(table_hbm.at[idx_v.at[pl.ds(b * CHUNK, CHUNK)]], rows[b], gs[b])

        ngrp = n_per_w // NBUF

        def body(g, carry):
            for b in range(NBUF):
                j = g * NBUF + b
                bp = (b + NBUF - 1) % NBUF
                pltpu.make_async_copy(
                    table_hbm.at[idx_v.at[pl.ds(j * CHUNK, CHUNK)]], rows[b], gs[b]).wait()
                pltpu.async_copy(rows[b], out_slice(j), ws[b])

                @pl.when(j + NBUF - 1 < n_per_w)
                def _():
                    @pl.when(j >= 1)
                    def _w():
                        pltpu.make_async_copy(
                            rows[bp], out_slice(j - 1), ws[bp]).wait()

                    pltpu.async_copy(
                        table_hbm.at[idx_v.at[pl.ds((j + NBUF - 1) * CHUNK, CHUNK)]], rows[bp], gs[bp])

            return carry

        lax.fori_loop(0, ngrp, body, 0)

        for b in range(NBUF):
            j = n_per_w - NBUF + b
            pltpu.make_async_copy(rows[b], out_slice(j), ws[b]).wait()

    return k


def kernel(input, table):
    flat = input.reshape(-1).astype(jnp.int32)
    n_chunks = flat.shape[0] // CHUNK
    out = _gather_fn(n_chunks // NW)(flat, table)
    return out.reshape(input.shape + (DIM,))


# speedup vs baseline: 1.7814x; 1.7790x over previous
import functools

import jax
import jax.numpy as jnp
from jax import lax
from jax.experimental import pallas as pl
from jax.experimental.pallas import tpu as pltpu
from jax.experimental.pallas import tpu_sc as plsc

DIM = 128
HIST = 50
PITCH = 64
NC, NS = 2, 16
NW = NC * NS


@functools.lru_cache(maxsize=None)
def _gather_fn(batch: int):
    rows_w = batch // NW
    nidx = rows_w * HIST
    mesh = plsc.VectorSubcoreMesh(core_axis_name="c", subcore_axis_name="s")
    NBUF = 4

    @functools.partial(
        pl.kernel,
        mesh=mesh,
        out_type=jax.ShapeDtypeStruct((batch, HIST, DIM), jnp.float32),
        compiler_params=pltpu.CompilerParams(use_tc_tiling_on_sc=True),
        scratch_types=[
            pltpu.VMEM((nidx + PITCH,), jnp.int32),
            pltpu.VMEM((rows_w, PITCH), jnp.int32),
        ]
        + [pltpu.VMEM((HIST, DIM), jnp.float32)] * NBUF
        + [pltpu.SemaphoreType.DMA] * (2 * NBUF),
    )
    def k(idx_hbm, table_hbm, out_hbm, idx_raw, idx_pad, *bufs):
        rows = bufs[:NBUF]
        gs = bufs[NBUF:2 * NBUF]
        ws = bufs[2 * NBUF:]
        wid = lax.axis_index("s") * NC + lax.axis_index("c")
        rbase = wid * rows_w
        pltpu.sync_copy(idx_hbm.at[pl.ds(rbase * HIST, nidx)],
                        idx_raw.at[pl.ds(0, nidx)])

        def rp(r, carry):
            for o in (0, 16, 32, 48):
                idx_pad[r, pl.ds(o, 16)] = idx_raw[pl.ds(r * HIST + o, 16)]
            return carry

        lax.fori_loop(0, rows_w, rp, 0)

        def gather_start(j, b):
            pltpu.async_copy(
                table_hbm.at[idx_pad.at[j, pl.ds(0, HIST)]], rows[b], gs[b])

        def gather_wait(j, b):
            pltpu.make_async_copy(
                table_hbm.at[idx_pad.at[j, pl.ds(0, HIST)]], rows[b], gs[b]).wait()

        def write_start(j, b):
            pltpu.async_copy(rows[b], out_hbm.at[rbase + j], ws[b])

        def write_wait(j, b):
            pltpu.make_async_copy(rows[b], out_hbm.at[rbase + j], ws[b]).wait()

        for b in range(NBUF - 1):
            gather_start(b, b)

        def body(g, carry):
            for b in range(NBUF):
                j = g * NBUF + b
                bp = (b + NBUF - 1) % NBUF
                gather_wait(j, b)
                write_start(j, b)

                @pl.when(j + NBUF - 1 < rows_w)
                def _():
                    @pl.when(j >= 1)
                    def _w():
                        write_wait(j - 1, bp)

                    gather_start(j + NBUF - 1, bp)

            return carry

        lax.fori_loop(0, rows_w // NBUF, body, 0)

        for b in range(NBUF):
            write_wait(rows_w - NBUF + b, b)

    return k


def kernel(input, table):
    flat = input.reshape(-1).astype(jnp.int32)
    out = _gather_fn(input.shape[0])(flat, table)
    return out.reshape(input.shape + (DIM,))


# speedup vs baseline: 3.4497x; 1.9366x over previous
import functools

import jax
import jax.numpy as jnp
from jax import lax
from jax.experimental import pallas as pl
from jax.experimental.pallas import tpu as pltpu
from jax.experimental.pallas import tpu_sc as plsc

DIM = 128
CHUNK = 128
NC, NS = 2, 16
NW = NC * NS


@functools.lru_cache(maxsize=None)
def _gather_fn(n_per_w: int):
    mesh = plsc.VectorSubcoreMesh(core_axis_name="c", subcore_axis_name="s")
    NBUF = 4

    @functools.partial(
        pl.kernel,
        mesh=mesh,
        out_type=jax.ShapeDtypeStruct((NW * n_per_w * CHUNK, DIM), jnp.float32),
        compiler_params=pltpu.CompilerParams(use_tc_tiling_on_sc=True),
        scratch_types=[pltpu.VMEM((n_per_w * CHUNK,), jnp.int32)]
        + [pltpu.VMEM((CHUNK, DIM), jnp.float32)] * NBUF
        + [pltpu.SemaphoreType.DMA] * (2 * NBUF),
    )
    def k(idx_hbm, table_hbm, out_hbm, idx_v, *bufs):
        rows = bufs[:NBUF]
        gs = bufs[NBUF:2 * NBUF]
        ws = bufs[2 * NBUF:]
        wid = lax.axis_index("s") * NC + lax.axis_index("c")
        cbase = wid * n_per_w
        pltpu.sync_copy(idx_hbm.at[pl.ds(cbase * CHUNK, n_per_w * CHUNK)], idx_v)

        def gather_start(j, b):
            pltpu.async_copy(
                table_hbm.at[idx_v.at[pl.ds(j * CHUNK, CHUNK)]], rows[b], gs[b])

        def gather_wait(j, b):
            pltpu.make_async_copy(
                table_hbm.at[idx_v.at[pl.ds(j * CHUNK, CHUNK)]], rows[b],
                gs[b]).wait()

        def write_start(j, b):
            pltpu.async_copy(
                rows[b], out_hbm.at[pl.ds((cbase + j) * CHUNK, CHUNK)], ws[b])

        def write_wait(j, b):
            pltpu.make_async_copy(
                rows[b], out_hbm.at[pl.ds((cbase + j) * CHUNK, CHUNK)],
                ws[b]).wait()

        for b in range(NBUF - 1):
            gather_start(b, b)

        def body(g, carry):
            for b in range(NBUF):
                j = g * NBUF + b
                bp = (b + NBUF - 1) % NBUF
                gather_wait(j, b)
                write_start(j, b)

                @pl.when(j + NBUF - 1 < n_per_w)
                def _():
                    @pl.when(j >= 1)
                    def _w():
                        write_wait(j - 1, bp)

                    gather_start(j + NBUF - 1, bp)

            return carry

        lax.fori_loop(0, n_per_w // NBUF, body, 0)

        for b in range(NBUF):
            write_wait(n_per_w - NBUF + b, b)

    return k


def kernel(input, table):
    batch, hist = input.shape
    flat_t = jnp.transpose(input).reshape(-1).astype(jnp.int32)
    n_chunks = flat_t.shape[0] // CHUNK
    out = _gather_fn(n_chunks // NW)(flat_t, table)
    out = out.reshape(hist, batch, DIM)
    return jnp.transpose(out, (1, 0, 2))
